# Initial kernel scaffold; baseline (speedup 1.0000x reference)
import jax
import jax.numpy as jnp
from jax.experimental import pallas as pl


def kernel(input_data, batch_sizes, h0, c0, mask_x, mask_h, W_ih, W_hh, b_ih, b_hh):
    raise NotImplementedError("write your pallas kernel here")


# speedup vs baseline: 18.4632x; 18.4632x over previous
import jax
import jax.numpy as jnp
from jax.experimental import pallas as pl
from jax.experimental.pallas import tpu as pltpu

_BATCH = 16
_MAXLEN = 512
_D = 256
_H = 256
_WIN = 24


def _lstm_loop_kernel(starts_ref, sizes_ref, x_ref, h0_ref, c0_ref,
                      mx_ref, mh_ref, wih_ref, whh_ref, b_ref,
                      out_ref, hn_ref, cn_ref, h_scr, c_scr):
    h_scr[...] = h0_ref[...]
    c_scr[...] = c0_ref[...]
    mx = mx_ref[...]
    mh = mh_ref[...]
    wih = wih_ref[...]
    whh = whh_ref[...]
    b = b_ref[...]
    row = jax.lax.broadcasted_iota(jnp.int32, (_BATCH, 1), 0)
    row24 = jax.lax.broadcasted_iota(jnp.int32, (_WIN, 1), 0)
    zpad = jnp.zeros((_WIN - _BATCH, _H), jnp.float32)

    def body(t, _):
        start = starts_ref[t]
        base = pl.multiple_of((start // 8) * 8, 8)
        off = start - base
        win = x_ref[pl.ds(base, _WIN), :]
        x = pltpu.roll(win, jax.lax.rem(_WIN - off, _WIN), axis=0)[:_BATCH]
        h = h_scr[...]
        c = c_scr[...]
        gates = (jnp.dot(x * mx, wih, preferred_element_type=jnp.float32)
                 + jnp.dot(h * mh, whh, preferred_element_type=jnp.float32)
                 + b)
        i = jax.nn.sigmoid(gates[:, :_H])
        f = jax.nn.sigmoid(gates[:, _H:2 * _H])
        g = jnp.tanh(gates[:, 2 * _H:3 * _H])
        o = jax.nn.sigmoid(gates[:, 3 * _H:])
        c2 = f * c + i * g
        h2 = o * jnp.tanh(c2)

        old = out_ref[pl.ds(base, _WIN), :]
        new = pltpu.roll(jnp.concatenate([h2, zpad], axis=0), off, axis=0)
        out_ref[pl.ds(base, _WIN), :] = jnp.where(row24 >= off, new, old)

        size = sizes_ref[t]
        act = row < size
        h_scr[...] = jnp.where(act, h2, h)
        c_scr[...] = jnp.where(act, c2, c)
        return 0

    jax.lax.fori_loop(0, _MAXLEN, body, 0, unroll=False)
    hn_ref[...] = h_scr[...]
    cn_ref[...] = c_scr[...]


def kernel(input_data, batch_sizes, h0, c0, mask_x, mask_h, W_ih, W_hh, b_ih, b_hh):
    total = input_data.shape[0]
    sizes = batch_sizes.astype(jnp.int32)
    starts = jnp.cumsum(sizes) - sizes
    x_pad = jnp.pad(input_data, ((0, _BATCH), (0, 0)))
    b = (b_ih + b_hh).reshape(1, 4 * _H)

    out_pad, hn, cn = pl.pallas_call(
        _lstm_loop_kernel,
        out_shape=[
            jax.ShapeDtypeStruct((total + _BATCH, _H), jnp.float32),
            jax.ShapeDtypeStruct((_BATCH, _H), jnp.float32),
            jax.ShapeDtypeStruct((_BATCH, _H), jnp.float32),
        ],
        in_specs=[
            pl.BlockSpec(memory_space=pltpu.SMEM),
            pl.BlockSpec(memory_space=pltpu.SMEM),
            pl.BlockSpec(memory_space=pltpu.VMEM),
            pl.BlockSpec(memory_space=pltpu.VMEM),
            pl.BlockSpec(memory_space=pltpu.VMEM),
            pl.BlockSpec(memory_space=pltpu.VMEM),
            pl.BlockSpec(memory_space=pltpu.VMEM),
            pl.BlockSpec(memory_space=pltpu.VMEM),
            pl.BlockSpec(memory_space=pltpu.VMEM),
            pl.BlockSpec(memory_space=pltpu.VMEM),
        ],
        out_specs=[
            pl.BlockSpec(memory_space=pltpu.VMEM),
            pl.BlockSpec(memory_space=pltpu.VMEM),
            pl.BlockSpec(memory_space=pltpu.VMEM),
        ],
        scratch_shapes=[
            pltpu.VMEM((_BATCH, _H), jnp.float32),
            pltpu.VMEM((_BATCH, _H), jnp.float32),
        ],
    )(starts, sizes, x_pad, h0, c0, mask_x, mask_h, W_ih.T, W_hh.T, b)

    return out_pad[:total], hn, cn
